# baseline (speedup 1.0000x reference)
import functools

import jax
import jax.numpy as jnp
from jax import lax
from jax.experimental import pallas as pl
from jax.experimental.pallas import tpu as pltpu
from jax.experimental.pallas import tpu_sc as plsc

N_NODES = 100000
N_EDGES = 1600000
EMB = 16
HID = 32
NCLS = 10
NG = 256

NPAD = 100352
NHALF = NPAD // 2
X2 = 2 * NPAD
EPAD = 1605632
EPT = EPAD // 16
NCHUNK = EPT // 128
BLK_E = 2048
ACC_ROWS = 50304
ACC2 = 100480
RPT2 = NPAD // 16
DEG_ROWS = 3200
RPT = NHALF // 16
DPT = DEG_ROWS // 16 - 4
DHALF = NHALF // 16

_MESH = plsc.VectorSubcoreMesh(
    core_axis_name="c", subcore_axis_name="s", num_cores=2, num_subcores=16)

def _embed_body(x0_ref, x1_ref, se_ref, ce_ref, out_ref):
    col = lax.broadcasted_iota(jnp.int32, (BLK_E, EMB), 1)
    oh0 = (col == x0_ref[...]).astype(jnp.float32)
    oh1 = (col == x1_ref[...]).astype(jnp.float32)
    out_ref[0] = lax.dot_general(oh0, se_ref[...], (((1,), (0,)), ((), ())),
                                 preferred_element_type=jnp.float32)
    out_ref[1] = lax.dot_general(oh1, ce_ref[...], (((1,), (0,)), ((), ())),
                                 preferred_element_type=jnp.float32)


def _embed(x0, x1, se, ce):
    return pl.pallas_call(
        _embed_body,
        grid=(NPAD // BLK_E,),
        in_specs=[
            pl.BlockSpec((BLK_E, 1), lambda i: (i, 0)),
            pl.BlockSpec((BLK_E, 1), lambda i: (i, 0)),
            pl.BlockSpec((EMB, EMB), lambda i: (0, 0)),
            pl.BlockSpec((EMB, EMB), lambda i: (0, 0)),
        ],
        out_specs=pl.BlockSpec((2, BLK_E, EMB), lambda i: (0, i, 0)),
        out_shape=jax.ShapeDtypeStruct((2, NPAD, EMB), jnp.float32),
    )(x0, x1, se, ce)


def _deg_body(dst_hbm, deg_hbm, dstb, sidx, hist, zbuf16, deg_sh):
    c = lax.axis_index("c")
    s = lax.axis_index("s")
    base_node = c * NHALF
    lane = lax.broadcasted_iota(jnp.int32, (16,), 0)
    zero16 = jnp.zeros((16,), jnp.float32)
    ones16 = jnp.ones((16,), jnp.float32)

    @pl.loop(0, 128)
    def _(r):
        zbuf16[r, pl.ds(0, 16)] = zero16

    @pl.loop(0, DEG_ROWS)
    def _(r):
        hist[r, pl.ds(0, 16)] = zero16

    @pl.loop(s, DEG_ROWS // 128, step=16)
    def _(k):
        pltpu.sync_copy(zbuf16, deg_sh.at[pl.ds(k * 128, 128)])

    plsc.subcore_barrier()

    ebase = s * EPT
    garb = NHALF + lane

    @pl.loop(0, NCHUNK)
    def _(k):
        @pl.when(k % 16 == 0)
        def _():
            blk = ebase + (k // 16) * BLK_E
            pltpu.sync_copy(dst_hbm.at[pl.ds(blk, BLK_E)], dstb)

        q = (k % 16) * 128
        for j in range(8):
            d = dstb[pl.ds(q + j * 16, 16)]
            local = d - base_node
            ok = plsc.bitcast(local, jnp.uint32) < jnp.uint32(NHALF)
            idx = jnp.where(ok, local, garb)
            plsc.addupdate_scatter(hist, [idx >> 4, idx & 15], ones16)

    @pl.loop(0, DEG_ROWS // 128)
    def _(m):
        for j in range(8):
            sidx[pl.ds(j * 16, 16)] = m * 128 + j * 16 + lane
        pltpu.sync_copy(hist.at[pl.ds(m * 128, 128)], deg_sh.at[sidx],
                        add=True)

    plsc.subcore_barrier()

    pltpu.sync_copy(deg_sh.at[pl.ds(s * DPT, DPT)],
                    deg_hbm.at[pl.ds(c * DHALF + s * DPT, DPT)])


_deg = pl.kernel(
    _deg_body,
    out_type=jax.ShapeDtypeStruct((NPAD // 16, 16), jnp.float32),
    mesh=_MESH,
    compiler_params=pltpu.CompilerParams(use_tc_tiling_on_sc=False, needs_layout_passes=False),
    scratch_types=[
        pltpu.VMEM((BLK_E,), jnp.int32),
        pltpu.VMEM((128,), jnp.int32),
        pltpu.VMEM((DEG_ROWS, 16), jnp.float32),
        pltpu.VMEM((128, 16), jnp.float32),
        pltpu.VMEM_SHARED((DEG_ROWS, 16), jnp.float32),
    ],
)


def _agg_body(h_hbm, src_hbm, dst_hbm, agg_hbm,
              srcb, dstb, sidx, rows0, rows1, zbuf, gsem0, gsem1, acc_sh):
    c = lax.axis_index("c")
    s = lax.axis_index("s")
    coff = c * NPAD
    lane = lax.broadcasted_iota(jnp.int32, (16,), 0)
    zero16 = jnp.zeros((16,), jnp.float32)

    @pl.loop(0, 128)
    def _(r):
        zbuf[r, pl.ds(0, 16)] = zero16

    @pl.loop(s, ACC2 // 128, step=16)
    def _(k):
        pltpu.sync_copy(zbuf, acc_sh.at[pl.ds(k * 128, 128)])

    plsc.subcore_barrier()

    ebase = s * EPT
    garb = NPAD + lane
    npairs = NCHUNK // 2

    def stage(b, buf):
        blk = ebase + b * BLK_E
        pltpu.sync_copy(src_hbm.at[pl.ds(blk, BLK_E)], srcb.at[buf])
        pltpu.sync_copy(dst_hbm.at[pl.ds(blk, BLK_E)], dstb.at[buf])

        @pl.loop(0, BLK_E // 16)
        def _(j):
            srcb[buf, pl.ds(j * 16, 16)] = (
                srcb[buf, pl.ds(j * 16, 16)] + coff)

    def gather(k, rbuf, sem):
        b = k // 16
        q = (k % 16) * 128
        return pltpu.async_copy(
            h_hbm.at[srcb.at[b % 2, pl.ds(q, 128)]], rbuf, sem)

    def make_sidx(k):
        b = k // 16
        q = (k % 16) * 128
        for j in range(8):
            d = dstb[b % 2, pl.ds(q + j * 16, 16)]
            ok = plsc.bitcast(d, jnp.uint32) < jnp.uint32(NPAD)
            sidx[pl.ds(j * 16, 16)] = jnp.where(ok, d, garb)

    stage(0, 0)
    g0 = gather(0, rows0, gsem0)

    @pl.loop(0, npairs)
    def _(p):
        k0 = 2 * p
        k1 = k0 + 1

        @pl.when((p % 8 == 6) & (p < npairs - 8))
        def _():
            b = p // 8 + 1
            stage(b, b % 2)

        make_sidx(k0)
        gather(k1, rows1, gsem1)
        g0w = pltpu.make_async_copy(
            h_hbm.at[srcb.at[(k0 // 16) % 2, pl.ds((k0 % 16) * 128, 128)]],
            rows0, gsem0)
        g0w.wait()
        pltpu.sync_copy(rows0, acc_sh.at[sidx], add=True)

        make_sidx(k1)

        @pl.when(p + 1 < npairs)
        def _():
            gather(k0 + 2, rows0, gsem0)

        g1w = pltpu.make_async_copy(
            h_hbm.at[srcb.at[(k1 // 16) % 2, pl.ds((k1 % 16) * 128, 128)]],
            rows1, gsem1)
        g1w.wait()
        pltpu.sync_copy(rows1, acc_sh.at[sidx], add=True)

    plsc.subcore_barrier()

    pltpu.sync_copy(acc_sh.at[pl.ds(s * RPT2, RPT2)],
                    agg_hbm.at[pl.ds(c * NPAD + s * RPT2, RPT2)])


_agg = pl.kernel(
    _agg_body,
    out_type=jax.ShapeDtypeStruct((X2, EMB), jnp.float32),
    mesh=_MESH,
    compiler_params=pltpu.CompilerParams(use_tc_tiling_on_sc=False, needs_layout_passes=False),
    scratch_types=[
        pltpu.VMEM((2, BLK_E), jnp.int32),
        pltpu.VMEM((2, BLK_E), jnp.int32),
        pltpu.VMEM((128,), jnp.int32),
        pltpu.VMEM((128, EMB), jnp.float32),
        pltpu.VMEM((128, EMB), jnp.float32),
        pltpu.VMEM((128, EMB), jnp.float32),
        pltpu.SemaphoreType.DMA,
        pltpu.SemaphoreType.DMA,
        pltpu.VMEM_SHARED((ACC2, EMB), jnp.float32),
    ],
)


def _combine_body(agg_ref, h_ref, deg_ref, wl_ref, wr_ref, b_ref, out_ref):
    agg = jnp.concatenate([agg_ref[0], agg_ref[1]], axis=-1)
    h = jnp.concatenate([h_ref[0], h_ref[1]], axis=-1)
    inv = 1.0 / jnp.maximum(deg_ref[...], 1.0)
    aggm = agg * inv
    y = (lax.dot_general(aggm, wl_ref[...], (((1,), (1,)), ((), ())),
                         preferred_element_type=jnp.float32)
         + lax.dot_general(h, wr_ref[...], (((1,), (1,)), ((), ())),
                           preferred_element_type=jnp.float32)
         + b_ref[...])
    y = jnp.maximum(y, 0.0)
    out_ref[0] = y[:, :EMB]
    out_ref[1] = y[:, EMB:]


def _combine(agg, h, deg, Wl, Wr, b):
    return pl.pallas_call(
        _combine_body,
        grid=(NPAD // BLK_E,),
        in_specs=[
            pl.BlockSpec((2, BLK_E, EMB), lambda i: (0, i, 0)),
            pl.BlockSpec((2, BLK_E, EMB), lambda i: (0, i, 0)),
            pl.BlockSpec((BLK_E, 1), lambda i: (i, 0)),
            pl.BlockSpec((HID, HID), lambda i: (0, 0)),
            pl.BlockSpec((HID, HID), lambda i: (0, 0)),
            pl.BlockSpec((1, HID), lambda i: (0, 0)),
        ],
        out_specs=pl.BlockSpec((2, BLK_E, EMB), lambda i: (0, i, 0)),
        out_shape=jax.ShapeDtypeStruct((2, NPAD, EMB), jnp.float32),
    )(agg, h, deg, Wl, Wr, b)


def _cpool_body(agg_ref, h_ref, deg_ref, batch_ref, wl_ref, wr_ref, b_ref,
                wc_ref, bc_ref, out_ref, pooled, cnt):
    i = pl.program_id(0)

    @pl.when(i == 0)
    def _():
        pooled[...] = jnp.zeros_like(pooled)
        cnt[...] = jnp.zeros_like(cnt)

    agg = jnp.concatenate([agg_ref[0], agg_ref[1]], axis=-1)
    h = jnp.concatenate([h_ref[0], h_ref[1]], axis=-1)
    inv = 1.0 / jnp.maximum(deg_ref[...], 1.0)
    aggm = agg * inv
    h2 = jnp.maximum(
        lax.dot_general(aggm, wl_ref[...], (((1,), (1,)), ((), ())),
                        preferred_element_type=jnp.float32)
        + lax.dot_general(h, wr_ref[...], (((1,), (1,)), ((), ())),
                          preferred_element_type=jnp.float32)
        + b_ref[...], 0.0)

    oh = (lax.broadcasted_iota(jnp.int32, (NG, BLK_E), 0)
          == batch_ref[...]).astype(jnp.float32)
    pooled[...] += lax.dot_general(oh, h2, (((1,), (0,)), ((), ())),
                                   preferred_element_type=jnp.float32)
    cnt[...] += jnp.sum(oh, axis=1, keepdims=True)

    @pl.when(i == pl.num_programs(0) - 1)
    def _():
        pm = pooled[...] / jnp.maximum(cnt[...], 1.0)
        out_ref[...] = (lax.dot_general(pm, wc_ref[...],
                                        (((1,), (1,)), ((), ())),
                                        preferred_element_type=jnp.float32)
                        + bc_ref[...])


def _cpool(agg, h, deg, batch2d, Wl, Wr, b, Wc, bc):
    return pl.pallas_call(
        _cpool_body,
        grid=(NPAD // BLK_E,),
        in_specs=[
            pl.BlockSpec((2, BLK_E, EMB), lambda i: (0, i, 0)),
            pl.BlockSpec((2, BLK_E, EMB), lambda i: (0, i, 0)),
            pl.BlockSpec((BLK_E, 1), lambda i: (i, 0)),
            pl.BlockSpec((1, BLK_E), lambda i: (0, i)),
            pl.BlockSpec((HID, HID), lambda i: (0, 0)),
            pl.BlockSpec((HID, HID), lambda i: (0, 0)),
            pl.BlockSpec((1, HID), lambda i: (0, 0)),
            pl.BlockSpec((NCLS, HID), lambda i: (0, 0)),
            pl.BlockSpec((1, NCLS), lambda i: (0, 0)),
        ],
        out_specs=pl.BlockSpec((NG, NCLS), lambda i: (0, 0)),
        out_shape=jax.ShapeDtypeStruct((NG, NCLS), jnp.float32),
        scratch_shapes=[
            pltpu.VMEM((NG, HID), jnp.float32),
            pltpu.VMEM((NG, 1), jnp.float32),
        ],
    )(agg, h, deg, batch2d, Wl, Wr, b, Wc, bc)


def kernel(x, edge_index, batch, shape_emb, color_emb,
           W1l, W1r, b1, W2l, W2r, b2, Wc, bc):
    x = x.astype(jnp.int32)
    src = edge_index[0].astype(jnp.int32)
    dst = edge_index[1].astype(jnp.int32)
    batch = batch.astype(jnp.int32)

    xp = jnp.zeros((NPAD, 2), jnp.int32).at[:N_NODES].set(x)
    srcp = jnp.concatenate([src, jnp.zeros((EPAD - N_EDGES,), jnp.int32)])
    dstp = jnp.concatenate(
        [dst, jnp.full((EPAD - N_EDGES,), 1 << 30, jnp.int32)])
    batchp = jnp.concatenate(
        [batch, jnp.full((NPAD - N_NODES,), -1, jnp.int32)]).reshape(1, NPAD)
    deg = _deg(dstp)
    degc = deg.reshape(NPAD, 1)
    h0 = _embed(xp[:, :1], xp[:, 1:], shape_emb, color_emb)
    agg1 = _agg(h0.reshape(X2, EMB), srcp, dstp).reshape(2, NPAD, EMB)
    h1 = _combine(agg1, h0, degc, W1l, W1r, b1.reshape(1, HID))
    agg2 = _agg(h1.reshape(X2, EMB), srcp, dstp).reshape(2, NPAD, EMB)
    return _cpool(agg2, h1, degc, batchp, W2l, W2r, b2.reshape(1, HID),
                  Wc, bc.reshape(1, NCLS))


# speedup vs baseline: 1.0037x; 1.0037x over previous
import functools

import jax
import jax.numpy as jnp
from jax import lax
from jax.experimental import pallas as pl
from jax.experimental.pallas import tpu as pltpu
from jax.experimental.pallas import tpu_sc as plsc

N_NODES = 100000
N_EDGES = 1600000
EMB = 16
HID = 32
NCLS = 10
NG = 256

NPAD = 100352
NHALF = NPAD // 2
X2 = 2 * NPAD
EPAD = 1605632
EPT = EPAD // 16
NCHUNK = EPT // 128
BLK_E = 2048
ACC_ROWS = 50304
ACC2 = 100480
RPT2 = NPAD // 16
DEG_ROWS = 3200
RPT = NHALF // 16
DPT = DEG_ROWS // 16 - 4
DHALF = NHALF // 16

_MESH = plsc.VectorSubcoreMesh(
    core_axis_name="c", subcore_axis_name="s", num_cores=2, num_subcores=16)

def _embed_body(x0_ref, x1_ref, se_ref, ce_ref, outn_ref, outs_ref):
    col = lax.broadcasted_iota(jnp.int32, (BLK_E, EMB), 1)
    oh0 = (col == x0_ref[...]).astype(jnp.float32)
    oh1 = (col == x1_ref[...]).astype(jnp.float32)
    e0 = lax.dot_general(oh0, se_ref[...], (((1,), (0,)), ((), ())),
                         preferred_element_type=jnp.float32)
    e1 = lax.dot_general(oh1, ce_ref[...], (((1,), (0,)), ((), ())),
                         preferred_element_type=jnp.float32)
    outn_ref[...] = jnp.concatenate([e0, e1], axis=-1)
    outs_ref[0] = e0
    outs_ref[1] = e1


def _embed(x0, x1, se, ce):
    return pl.pallas_call(
        _embed_body,
        grid=(NPAD // BLK_E,),
        in_specs=[
            pl.BlockSpec((BLK_E, 1), lambda i: (i, 0)),
            pl.BlockSpec((BLK_E, 1), lambda i: (i, 0)),
            pl.BlockSpec((EMB, EMB), lambda i: (0, 0)),
            pl.BlockSpec((EMB, EMB), lambda i: (0, 0)),
        ],
        out_specs=[
            pl.BlockSpec((BLK_E, HID), lambda i: (i, 0)),
            pl.BlockSpec((2, BLK_E, EMB), lambda i: (0, i, 0)),
        ],
        out_shape=[
            jax.ShapeDtypeStruct((NPAD, HID), jnp.float32),
            jax.ShapeDtypeStruct((2, NPAD, EMB), jnp.float32),
        ],
    )(x0, x1, se, ce)


def _deg_body(dst_hbm, deg_hbm, dstb, sidx, hist, zbuf16, deg_sh):
    c = lax.axis_index("c")
    s = lax.axis_index("s")
    base_node = c * NHALF
    lane = lax.broadcasted_iota(jnp.int32, (16,), 0)
    zero16 = jnp.zeros((16,), jnp.float32)
    ones16 = jnp.ones((16,), jnp.float32)

    @pl.loop(0, 128)
    def _(r):
        zbuf16[r, pl.ds(0, 16)] = zero16

    @pl.loop(0, DEG_ROWS)
    def _(r):
        hist[r, pl.ds(0, 16)] = zero16

    @pl.loop(s, DEG_ROWS // 128, step=16)
    def _(k):
        pltpu.sync_copy(zbuf16, deg_sh.at[pl.ds(k * 128, 128)])

    plsc.subcore_barrier()

    ebase = s * EPT
    garb = NHALF + lane

    @pl.loop(0, NCHUNK)
    def _(k):
        @pl.when(k % 16 == 0)
        def _():
            blk = ebase + (k // 16) * BLK_E
            pltpu.sync_copy(dst_hbm.at[pl.ds(blk, BLK_E)], dstb)

        q = (k % 16) * 128
        for j in range(8):
            d = dstb[pl.ds(q + j * 16, 16)]
            local = d - base_node
            ok = plsc.bitcast(local, jnp.uint32) < jnp.uint32(NHALF)
            idx = jnp.where(ok, local, garb)
            plsc.addupdate_scatter(hist, [idx >> 4, idx & 15], ones16)

    @pl.loop(0, DEG_ROWS // 128)
    def _(m):
        for j in range(8):
            sidx[pl.ds(j * 16, 16)] = m * 128 + j * 16 + lane
        pltpu.sync_copy(hist.at[pl.ds(m * 128, 128)], deg_sh.at[sidx],
                        add=True)

    plsc.subcore_barrier()

    pltpu.sync_copy(deg_sh.at[pl.ds(s * DPT, DPT)],
                    deg_hbm.at[pl.ds(c * DHALF + s * DPT, DPT)])


_deg = pl.kernel(
    _deg_body,
    out_type=jax.ShapeDtypeStruct((NPAD // 16, 16), jnp.float32),
    mesh=_MESH,
    compiler_params=pltpu.CompilerParams(use_tc_tiling_on_sc=False, needs_layout_passes=False),
    scratch_types=[
        pltpu.VMEM((BLK_E,), jnp.int32),
        pltpu.VMEM((128,), jnp.int32),
        pltpu.VMEM((DEG_ROWS, 16), jnp.float32),
        pltpu.VMEM((128, 16), jnp.float32),
        pltpu.VMEM_SHARED((DEG_ROWS, 16), jnp.float32),
    ],
)


def _agg_body(h_hbm, src_hbm, dst_hbm, agg_hbm,
              srcb, dstb, sidx, rows0, rows1, zbuf, gsem0, gsem1, acc_sh):
    c = lax.axis_index("c")
    s = lax.axis_index("s")
    coff = c * NPAD
    lane = lax.broadcasted_iota(jnp.int32, (16,), 0)
    zero16 = jnp.zeros((16,), jnp.float32)

    @pl.loop(0, 128)
    def _(r):
        zbuf[r, pl.ds(0, 16)] = zero16

    @pl.loop(s, ACC2 // 128, step=16)
    def _(k):
        pltpu.sync_copy(zbuf, acc_sh.at[pl.ds(k * 128, 128)])

    plsc.subcore_barrier()

    ebase = s * EPT
    garb = NPAD + lane
    npairs = NCHUNK // 2

    def stage(b, buf):
        blk = ebase + b * BLK_E
        pltpu.sync_copy(src_hbm.at[pl.ds(blk, BLK_E)], srcb.at[buf])
        pltpu.sync_copy(dst_hbm.at[pl.ds(blk, BLK_E)], dstb.at[buf])

        @pl.loop(0, BLK_E // 16)
        def _(j):
            srcb[buf, pl.ds(j * 16, 16)] = (
                srcb[buf, pl.ds(j * 16, 16)] + coff)

    def gather(k, rbuf, sem):
        b = k // 16
        q = (k % 16) * 128
        return pltpu.async_copy(
            h_hbm.at[srcb.at[b % 2, pl.ds(q, 128)]], rbuf, sem)

    def make_sidx(k):
        b = k // 16
        q = (k % 16) * 128
        for j in range(8):
            d = dstb[b % 2, pl.ds(q + j * 16, 16)]
            ok = plsc.bitcast(d, jnp.uint32) < jnp.uint32(NPAD)
            sidx[pl.ds(j * 16, 16)] = jnp.where(ok, d, garb)

    stage(0, 0)
    g0 = gather(0, rows0, gsem0)

    @pl.loop(0, npairs)
    def _(p):
        k0 = 2 * p
        k1 = k0 + 1

        @pl.when((p % 8 == 6) & (p < npairs - 8))
        def _():
            b = p // 8 + 1
            stage(b, b % 2)

        make_sidx(k0)
        gather(k1, rows1, gsem1)
        g0w = pltpu.make_async_copy(
            h_hbm.at[srcb.at[(k0 // 16) % 2, pl.ds((k0 % 16) * 128, 128)]],
            rows0, gsem0)
        g0w.wait()
        pltpu.sync_copy(rows0, acc_sh.at[sidx], add=True)

        make_sidx(k1)

        @pl.when(p + 1 < npairs)
        def _():
            gather(k0 + 2, rows0, gsem0)

        g1w = pltpu.make_async_copy(
            h_hbm.at[srcb.at[(k1 // 16) % 2, pl.ds((k1 % 16) * 128, 128)]],
            rows1, gsem1)
        g1w.wait()
        pltpu.sync_copy(rows1, acc_sh.at[sidx], add=True)

    plsc.subcore_barrier()

    pltpu.sync_copy(acc_sh.at[pl.ds(s * RPT2, RPT2)],
                    agg_hbm.at[pl.ds(s * RPT2, RPT2), pl.ds(c * EMB, EMB)])


_agg = pl.kernel(
    _agg_body,
    out_type=jax.ShapeDtypeStruct((NPAD, HID), jnp.float32),
    mesh=_MESH,
    compiler_params=pltpu.CompilerParams(use_tc_tiling_on_sc=False, needs_layout_passes=False),
    scratch_types=[
        pltpu.VMEM((2, BLK_E), jnp.int32),
        pltpu.VMEM((2, BLK_E), jnp.int32),
        pltpu.VMEM((128,), jnp.int32),
        pltpu.VMEM((128, EMB), jnp.float32),
        pltpu.VMEM((128, EMB), jnp.float32),
        pltpu.VMEM((128, EMB), jnp.float32),
        pltpu.SemaphoreType.DMA,
        pltpu.SemaphoreType.DMA,
        pltpu.VMEM_SHARED((ACC2, EMB), jnp.float32),
    ],
)


def _combine_body(agg_ref, h_ref, deg_ref, wl_ref, wr_ref, b_ref,
                  outn_ref, outs_ref):
    inv = 1.0 / jnp.maximum(deg_ref[...], 1.0)
    aggm = agg_ref[...] * inv
    y = (lax.dot_general(aggm, wl_ref[...], (((1,), (1,)), ((), ())),
                         preferred_element_type=jnp.float32)
         + lax.dot_general(h_ref[...], wr_ref[...], (((1,), (1,)), ((), ())),
                           preferred_element_type=jnp.float32)
         + b_ref[...])
    y = jnp.maximum(y, 0.0)
    outn_ref[...] = y
    outs_ref[0] = y[:, :EMB]
    outs_ref[1] = y[:, EMB:]


def _combine(agg, h, deg, Wl, Wr, b):
    return pl.pallas_call(
        _combine_body,
        grid=(NPAD // BLK_E,),
        in_specs=[
            pl.BlockSpec((BLK_E, HID), lambda i: (i, 0)),
            pl.BlockSpec((BLK_E, HID), lambda i: (i, 0)),
            pl.BlockSpec((BLK_E, 1), lambda i: (i, 0)),
            pl.BlockSpec((HID, HID), lambda i: (0, 0)),
            pl.BlockSpec((HID, HID), lambda i: (0, 0)),
            pl.BlockSpec((1, HID), lambda i: (0, 0)),
        ],
        out_specs=[
            pl.BlockSpec((BLK_E, HID), lambda i: (i, 0)),
            pl.BlockSpec((2, BLK_E, EMB), lambda i: (0, i, 0)),
        ],
        out_shape=[
            jax.ShapeDtypeStruct((NPAD, HID), jnp.float32),
            jax.ShapeDtypeStruct((2, NPAD, EMB), jnp.float32),
        ],
    )(agg, h, deg, Wl, Wr, b)


def _cpool_body(agg_ref, h_ref, deg_ref, batch_ref, wl_ref, wr_ref, b_ref,
                wc_ref, bc_ref, out_ref, pooled, cnt):
    i = pl.program_id(0)

    @pl.when(i == 0)
    def _():
        pooled[...] = jnp.zeros_like(pooled)
        cnt[...] = jnp.zeros_like(cnt)

    inv = 1.0 / jnp.maximum(deg_ref[...], 1.0)
    aggm = agg_ref[...] * inv
    h2 = jnp.maximum(
        lax.dot_general(aggm, wl_ref[...], (((1,), (1,)), ((), ())),
                        preferred_element_type=jnp.float32)
        + lax.dot_general(h_ref[...], wr_ref[...], (((1,), (1,)), ((), ())),
                          preferred_element_type=jnp.float32)
        + b_ref[...], 0.0)

    oh = (lax.broadcasted_iota(jnp.int32, (NG, BLK_E), 0)
          == batch_ref[...]).astype(jnp.float32)
    pooled[...] += lax.dot_general(oh, h2, (((1,), (0,)), ((), ())),
                                   preferred_element_type=jnp.float32)
    cnt[...] += jnp.sum(oh, axis=1, keepdims=True)

    @pl.when(i == pl.num_programs(0) - 1)
    def _():
        pm = pooled[...] / jnp.maximum(cnt[...], 1.0)
        out_ref[...] = (lax.dot_general(pm, wc_ref[...],
                                        (((1,), (1,)), ((), ())),
                                        preferred_element_type=jnp.float32)
                        + bc_ref[...])


def _cpool(agg, h, deg, batch2d, Wl, Wr, b, Wc, bc):
    return pl.pallas_call(
        _cpool_body,
        grid=(NPAD // BLK_E,),
        in_specs=[
            pl.BlockSpec((BLK_E, HID), lambda i: (i, 0)),
            pl.BlockSpec((BLK_E, HID), lambda i: (i, 0)),
            pl.BlockSpec((BLK_E, 1), lambda i: (i, 0)),
            pl.BlockSpec((1, BLK_E), lambda i: (0, i)),
            pl.BlockSpec((HID, HID), lambda i: (0, 0)),
            pl.BlockSpec((HID, HID), lambda i: (0, 0)),
            pl.BlockSpec((1, HID), lambda i: (0, 0)),
            pl.BlockSpec((NCLS, HID), lambda i: (0, 0)),
            pl.BlockSpec((1, NCLS), lambda i: (0, 0)),
        ],
        out_specs=pl.BlockSpec((NG, NCLS), lambda i: (0, 0)),
        out_shape=jax.ShapeDtypeStruct((NG, NCLS), jnp.float32),
        scratch_shapes=[
            pltpu.VMEM((NG, HID), jnp.float32),
            pltpu.VMEM((NG, 1), jnp.float32),
        ],
    )(agg, h, deg, batch2d, Wl, Wr, b, Wc, bc)


def kernel(x, edge_index, batch, shape_emb, color_emb,
           W1l, W1r, b1, W2l, W2r, b2, Wc, bc):
    x = x.astype(jnp.int32)
    src = edge_index[0].astype(jnp.int32)
    dst = edge_index[1].astype(jnp.int32)
    batch = batch.astype(jnp.int32)

    xp = jnp.zeros((NPAD, 2), jnp.int32).at[:N_NODES].set(x)
    srcp = jnp.concatenate([src, jnp.zeros((EPAD - N_EDGES,), jnp.int32)])
    dstp = jnp.concatenate(
        [dst, jnp.full((EPAD - N_EDGES,), 1 << 30, jnp.int32)])
    batchp = jnp.concatenate(
        [batch, jnp.full((NPAD - N_NODES,), -1, jnp.int32)]).reshape(1, NPAD)
    deg = _deg(dstp)
    degc = deg.reshape(NPAD, 1)
    h0n, h0s = _embed(xp[:, :1], xp[:, 1:], shape_emb, color_emb)
    agg1 = _agg(h0s.reshape(X2, EMB), srcp, dstp)
    h1n, h1s = _combine(agg1, h0n, degc, W1l, W1r, b1.reshape(1, HID))
    agg2 = _agg(h1s.reshape(X2, EMB), srcp, dstp)
    return _cpool(agg2, h1n, degc, batchp, W2l, W2r, b2.reshape(1, HID),
                  Wc, bc.reshape(1, NCLS))


# speedup vs baseline: 1.0417x; 1.0378x over previous
import functools

import jax
import jax.numpy as jnp
from jax import lax
from jax.experimental import pallas as pl
from jax.experimental.pallas import tpu as pltpu
from jax.experimental.pallas import tpu_sc as plsc

N_NODES = 100000
N_EDGES = 1600000
EMB = 16
HID = 32
NCLS = 10
NG = 256

NPAD = 100352
NHALF = NPAD // 2
X2 = 2 * NPAD
EPAD = 1605632
EPT = EPAD // 16
NCHUNK = EPT // 128
BLK_E = 2048
TBLK = 2048
PBLK = 2048
ACC_ROWS = 50304
ACC2 = 100480
RPT2 = NPAD // 16
DEG_ROWS = 3200
RPT = NHALF // 16
DPT = DEG_ROWS // 16 - 4
DHALF = NHALF // 16

_MESH = plsc.VectorSubcoreMesh(
    core_axis_name="c", subcore_axis_name="s", num_cores=2, num_subcores=16)

def _embed_body(x0_ref, x1_ref, se_ref, ce_ref, outn_ref, outs_ref):
    col = lax.broadcasted_iota(jnp.int32, (TBLK, EMB), 1)
    oh0 = (col == x0_ref[...]).astype(jnp.float32)
    oh1 = (col == x1_ref[...]).astype(jnp.float32)
    e0 = lax.dot_general(oh0, se_ref[...], (((1,), (0,)), ((), ())),
                         preferred_element_type=jnp.float32)
    e1 = lax.dot_general(oh1, ce_ref[...], (((1,), (0,)), ((), ())),
                         preferred_element_type=jnp.float32)
    outn_ref[...] = jnp.concatenate([e0, e1], axis=-1)
    outs_ref[0] = e0
    outs_ref[1] = e1


def _embed(x0, x1, se, ce):
    return pl.pallas_call(
        _embed_body,
        grid=(NPAD // TBLK,),
        in_specs=[
            pl.BlockSpec((TBLK, 1), lambda i: (i, 0)),
            pl.BlockSpec((TBLK, 1), lambda i: (i, 0)),
            pl.BlockSpec((EMB, EMB), lambda i: (0, 0)),
            pl.BlockSpec((EMB, EMB), lambda i: (0, 0)),
        ],
        out_specs=[
            pl.BlockSpec((TBLK, HID), lambda i: (i, 0)),
            pl.BlockSpec((2, TBLK, EMB), lambda i: (0, i, 0)),
        ],
        out_shape=[
            jax.ShapeDtypeStruct((NPAD, HID), jnp.float32),
            jax.ShapeDtypeStruct((2, NPAD, EMB), jnp.float32),
        ],
    )(x0, x1, se, ce)


def _deg_body(dst_hbm, deg_hbm, dstb, sidx, hist, zbuf16, deg_sh):
    c = lax.axis_index("c")
    s = lax.axis_index("s")
    base_node = c * NHALF
    lane = lax.broadcasted_iota(jnp.int32, (16,), 0)
    zero16 = jnp.zeros((16,), jnp.float32)
    ones16 = jnp.ones((16,), jnp.float32)

    @pl.loop(0, 128)
    def _(r):
        zbuf16[r, pl.ds(0, 16)] = zero16

    @pl.loop(0, DEG_ROWS)
    def _(r):
        hist[r, pl.ds(0, 16)] = zero16

    @pl.loop(s, DEG_ROWS // 128, step=16)
    def _(k):
        pltpu.sync_copy(zbuf16, deg_sh.at[pl.ds(k * 128, 128)])

    plsc.subcore_barrier()

    ebase = s * EPT
    garb = NHALF + lane

    @pl.loop(0, NCHUNK)
    def _(k):
        @pl.when(k % 16 == 0)
        def _():
            blk = ebase + (k // 16) * BLK_E
            pltpu.sync_copy(dst_hbm.at[pl.ds(blk, BLK_E)], dstb)

        q = (k % 16) * 128
        for j in range(8):
            d = dstb[pl.ds(q + j * 16, 16)]
            local = d - base_node
            ok = plsc.bitcast(local, jnp.uint32) < jnp.uint32(NHALF)
            idx = jnp.where(ok, local, garb)
            plsc.addupdate_scatter(hist, [idx >> 4, idx & 15], ones16)

    @pl.loop(0, DEG_ROWS // 128)
    def _(m):
        for j in range(8):
            sidx[pl.ds(j * 16, 16)] = m * 128 + j * 16 + lane
        pltpu.sync_copy(hist.at[pl.ds(m * 128, 128)], deg_sh.at[sidx],
                        add=True)

    plsc.subcore_barrier()

    pltpu.sync_copy(deg_sh.at[pl.ds(s * DPT, DPT)],
                    deg_hbm.at[pl.ds(c * DHALF + s * DPT, DPT)])


_deg = pl.kernel(
    _deg_body,
    out_type=jax.ShapeDtypeStruct((NPAD // 16, 16), jnp.float32),
    mesh=_MESH,
    compiler_params=pltpu.CompilerParams(use_tc_tiling_on_sc=False, needs_layout_passes=False),
    scratch_types=[
        pltpu.VMEM((BLK_E,), jnp.int32),
        pltpu.VMEM((128,), jnp.int32),
        pltpu.VMEM((DEG_ROWS, 16), jnp.float32),
        pltpu.VMEM((128, 16), jnp.float32),
        pltpu.VMEM_SHARED((DEG_ROWS, 16), jnp.float32),
    ],
)


def _agg_body(h_hbm, src_hbm, dst_hbm, agg_hbm,
              srcb, dstb, sidx, rows0, rows1, zbuf, gsem0, gsem1, acc_sh):
    c = lax.axis_index("c")
    s = lax.axis_index("s")
    lane = lax.broadcasted_iota(jnp.int32, (16,), 0)
    zero16 = jnp.zeros((16,), jnp.float32)

    @pl.loop(0, 128)
    def _(r):
        zbuf[r, pl.ds(0, 16)] = zero16

    @pl.loop(s, ACC2 // 128, step=16)
    def _(k):
        pltpu.sync_copy(zbuf, acc_sh.at[pl.ds(k * 128, 128)])

    plsc.subcore_barrier()

    ebase = s * EPT
    garb = NPAD + lane
    npairs = NCHUNK // 2

    def stage(b, buf):
        blk = ebase + b * BLK_E
        pltpu.sync_copy(src_hbm.at[pl.ds(blk, BLK_E)], srcb.at[buf])
        pltpu.sync_copy(dst_hbm.at[pl.ds(blk, BLK_E)], dstb.at[buf])

    def gather(k, rbuf, sem):
        b = k // 16
        q = (k % 16) * 128
        return pltpu.async_copy(
            h_hbm.at[c].at[srcb.at[b % 2, pl.ds(q, 128)]], rbuf, sem)

    def make_sidx(k):
        b = k // 16
        q = (k % 16) * 128
        for j in range(8):
            d = dstb[b % 2, pl.ds(q + j * 16, 16)]
            ok = plsc.bitcast(d, jnp.uint32) < jnp.uint32(NPAD)
            sidx[pl.ds(j * 16, 16)] = jnp.where(ok, d, garb)

    stage(0, 0)
    g0 = gather(0, rows0, gsem0)

    @pl.loop(0, npairs)
    def _(p):
        k0 = 2 * p
        k1 = k0 + 1

        @pl.when((p % 8 == 6) & (p < npairs - 8))
        def _():
            b = p // 8 + 1
            stage(b, b % 2)

        make_sidx(k0)
        gather(k1, rows1, gsem1)
        g0w = pltpu.make_async_copy(
            h_hbm.at[c].at[srcb.at[(k0 // 16) % 2,
                                   pl.ds((k0 % 16) * 128, 128)]],
            rows0, gsem0)
        g0w.wait()
        pltpu.sync_copy(rows0, acc_sh.at[sidx], add=True)

        make_sidx(k1)

        @pl.when(p + 1 < npairs)
        def _():
            gather(k0 + 2, rows0, gsem0)

        g1w = pltpu.make_async_copy(
            h_hbm.at[c].at[srcb.at[(k1 // 16) % 2,
                                   pl.ds((k1 % 16) * 128, 128)]],
            rows1, gsem1)
        g1w.wait()
        pltpu.sync_copy(rows1, acc_sh.at[sidx], add=True)

    plsc.subcore_barrier()

    pltpu.sync_copy(acc_sh.at[pl.ds(s * RPT2, RPT2)],
                    agg_hbm.at[pl.ds(s * RPT2, RPT2), pl.ds(c * EMB, EMB)])


_agg = pl.kernel(
    _agg_body,
    out_type=jax.ShapeDtypeStruct((NPAD, HID), jnp.float32),
    mesh=_MESH,
    compiler_params=pltpu.CompilerParams(use_tc_tiling_on_sc=False, needs_layout_passes=False),
    scratch_types=[
        pltpu.VMEM((2, BLK_E), jnp.int32),
        pltpu.VMEM((2, BLK_E), jnp.int32),
        pltpu.VMEM((128,), jnp.int32),
        pltpu.VMEM((128, EMB), jnp.float32),
        pltpu.VMEM((128, EMB), jnp.float32),
        pltpu.VMEM((128, EMB), jnp.float32),
        pltpu.SemaphoreType.DMA,
        pltpu.SemaphoreType.DMA,
        pltpu.VMEM_SHARED((ACC2, EMB), jnp.float32),
    ],
)


def _combine_body(agg_ref, h_ref, deg_ref, wl_ref, wr_ref, b_ref,
                  outn_ref, outs_ref):
    inv = 1.0 / jnp.maximum(deg_ref[...], 1.0)
    aggm = agg_ref[...] * inv
    y = (lax.dot_general(aggm, wl_ref[...], (((1,), (1,)), ((), ())),
                         preferred_element_type=jnp.float32)
         + lax.dot_general(h_ref[...], wr_ref[...], (((1,), (1,)), ((), ())),
                           preferred_element_type=jnp.float32)
         + b_ref[...])
    y = jnp.maximum(y, 0.0)
    outn_ref[...] = y
    outs_ref[0] = y[:, :EMB]
    outs_ref[1] = y[:, EMB:]


def _combine(agg, h, deg, Wl, Wr, b):
    return pl.pallas_call(
        _combine_body,
        grid=(NPAD // TBLK,),
        in_specs=[
            pl.BlockSpec((TBLK, HID), lambda i: (i, 0)),
            pl.BlockSpec((TBLK, HID), lambda i: (i, 0)),
            pl.BlockSpec((TBLK, 1), lambda i: (i, 0)),
            pl.BlockSpec((HID, HID), lambda i: (0, 0)),
            pl.BlockSpec((HID, HID), lambda i: (0, 0)),
            pl.BlockSpec((1, HID), lambda i: (0, 0)),
        ],
        out_specs=[
            pl.BlockSpec((TBLK, HID), lambda i: (i, 0)),
            pl.BlockSpec((2, TBLK, EMB), lambda i: (0, i, 0)),
        ],
        out_shape=[
            jax.ShapeDtypeStruct((NPAD, HID), jnp.float32),
            jax.ShapeDtypeStruct((2, NPAD, EMB), jnp.float32),
        ],
    )(agg, h, deg, Wl, Wr, b)


def _cpool_body(agg_ref, h_ref, deg_ref, batch_ref, wl_ref, wr_ref, b_ref,
                wc_ref, bc_ref, out_ref, pooled, cnt):
    i = pl.program_id(0)

    @pl.when(i == 0)
    def _():
        pooled[...] = jnp.zeros_like(pooled)
        cnt[...] = jnp.zeros_like(cnt)

    inv = 1.0 / jnp.maximum(deg_ref[...], 1.0)
    aggm = agg_ref[...] * inv
    h2 = jnp.maximum(
        lax.dot_general(aggm, wl_ref[...], (((1,), (1,)), ((), ())),
                        preferred_element_type=jnp.float32)
        + lax.dot_general(h_ref[...], wr_ref[...], (((1,), (1,)), ((), ())),
                          preferred_element_type=jnp.float32)
        + b_ref[...], 0.0)

    oh = (lax.broadcasted_iota(jnp.int32, (NG, PBLK), 0)
          == batch_ref[...]).astype(jnp.float32)
    pooled[...] += lax.dot_general(oh, h2, (((1,), (0,)), ((), ())),
                                   preferred_element_type=jnp.float32)
    cnt[...] += jnp.sum(oh, axis=1, keepdims=True)

    @pl.when(i == pl.num_programs(0) - 1)
    def _():
        pm = pooled[...] / jnp.maximum(cnt[...], 1.0)
        out_ref[...] = (lax.dot_general(pm, wc_ref[...],
                                        (((1,), (1,)), ((), ())),
                                        preferred_element_type=jnp.float32)
                        + bc_ref[...])


def _cpool(agg, h, deg, batch2d, Wl, Wr, b, Wc, bc):
    return pl.pallas_call(
        _cpool_body,
        grid=(NPAD // PBLK,),
        in_specs=[
            pl.BlockSpec((PBLK, HID), lambda i: (i, 0)),
            pl.BlockSpec((PBLK, HID), lambda i: (i, 0)),
            pl.BlockSpec((PBLK, 1), lambda i: (i, 0)),
            pl.BlockSpec((1, PBLK), lambda i: (0, i)),
            pl.BlockSpec((HID, HID), lambda i: (0, 0)),
            pl.BlockSpec((HID, HID), lambda i: (0, 0)),
            pl.BlockSpec((1, HID), lambda i: (0, 0)),
            pl.BlockSpec((NCLS, HID), lambda i: (0, 0)),
            pl.BlockSpec((1, NCLS), lambda i: (0, 0)),
        ],
        out_specs=pl.BlockSpec((NG, NCLS), lambda i: (0, 0)),
        out_shape=jax.ShapeDtypeStruct((NG, NCLS), jnp.float32),
        scratch_shapes=[
            pltpu.VMEM((NG, HID), jnp.float32),
            pltpu.VMEM((NG, 1), jnp.float32),
        ],
    )(agg, h, deg, batch2d, Wl, Wr, b, Wc, bc)


def kernel(x, edge_index, batch, shape_emb, color_emb,
           W1l, W1r, b1, W2l, W2r, b2, Wc, bc):
    x = x.astype(jnp.int32)
    src = edge_index[0].astype(jnp.int32)
    dst = edge_index[1].astype(jnp.int32)
    batch = batch.astype(jnp.int32)

    xp = jnp.zeros((NPAD, 2), jnp.int32).at[:N_NODES].set(x)
    srcp = jnp.concatenate([src, jnp.zeros((EPAD - N_EDGES,), jnp.int32)])
    dstp = jnp.concatenate(
        [dst, jnp.full((EPAD - N_EDGES,), 1 << 30, jnp.int32)])
    batchp = jnp.concatenate(
        [batch, jnp.full((NPAD - N_NODES,), -1, jnp.int32)]).reshape(1, NPAD)
    deg = _deg(dstp)
    degc = deg.reshape(NPAD, 1)
    h0n, h0s = _embed(xp[:, :1], xp[:, 1:], shape_emb, color_emb)
    agg1 = _agg(h0s, srcp, dstp)
    h1n, h1s = _combine(agg1, h0n, degc, W1l, W1r, b1.reshape(1, HID))
    agg2 = _agg(h1s, srcp, dstp)
    return _cpool(agg2, h1n, degc, batchp, W2l, W2r, b2.reshape(1, HID),
                  Wc, bc.reshape(1, NCLS))
